# Initial kernel scaffold; baseline (speedup 1.0000x reference)
import functools

import jax
import jax.numpy as jnp
from jax import lax
from jax.experimental import pallas as pl
from jax.experimental.pallas import tpu as pltpu
from jax.experimental.pallas import tpu_sc as plsc

VOCAB = 1000000
EMB_DIM = 32
BATCH = 16384
FIELDS = 26
TOTAL = BATCH * FIELDS

_info = plsc.get_sparse_core_info()
NC, NS = _info.num_cores, _info.num_subcores
NW = NC * NS

CHUNK = 128
PER_W = TOTAL // NW
NCHUNK = PER_W // CHUNK
NBUF = 2

assert PER_W % CHUNK == 0


def _body(ids_hbm, table_hbm, out_hbm, idx_v, rows_v, gsem):
    wid = lax.axis_index("s") * NC + lax.axis_index("c")
    chunk0 = wid * NCHUNK

    pltpu.sync_copy(ids_hbm.at[pl.ds(chunk0, NCHUNK)], idx_v)

    def start_gather(c, b):
        pltpu.async_copy(table_hbm.at[idx_v.at[c]], rows_v.at[b], gsem)

    for b in range(NBUF):
        start_gather(b, b)

    def group(g, with_next):
        for b in range(NBUF):
            c = g * NBUF + b
            pltpu.make_async_copy(table_hbm.at[idx_v.at[c]], rows_v.at[b],
                                  gsem).wait()
            pltpu.sync_copy(rows_v.at[b],
                            out_hbm.at[pl.ds((chunk0 + c) * CHUNK, CHUNK)])
            if with_next:
                start_gather(c + NBUF, b)

    pl.loop(0, NCHUNK // NBUF - 1)(lambda g: group(g, True))
    group(NCHUNK // NBUF - 1, False)


def kernel(ids, table):
    flat_ids = ids.reshape(-1).astype(jnp.int32)
    ids2d = flat_ids.reshape(TOTAL // CHUNK, CHUNK)

    mesh = plsc.VectorSubcoreMesh(core_axis_name="c", subcore_axis_name="s")
    out = pl.kernel(
        _body,
        out_type=jax.ShapeDtypeStruct((TOTAL, EMB_DIM), jnp.float32),
        mesh=mesh,
        scratch_types=[
            pltpu.VMEM((NCHUNK, CHUNK), jnp.int32),
            pltpu.VMEM((NBUF, CHUNK, EMB_DIM), jnp.float32),
            pltpu.SemaphoreType.DMA,
        ],
    )(ids2d, table)
    return out.reshape(BATCH, FIELDS, EMB_DIM)


# speedup vs baseline: 1.5231x; 1.5231x over previous
import functools

import jax
import jax.numpy as jnp
from jax import lax
from jax.experimental import pallas as pl
from jax.experimental.pallas import tpu as pltpu
from jax.experimental.pallas import tpu_sc as plsc

VOCAB = 1000000
EMB_DIM = 32
BATCH = 16384
FIELDS = 26
TOTAL = BATCH * FIELDS

_info = plsc.get_sparse_core_info()
NC, NS = _info.num_cores, _info.num_subcores
NW = NC * NS

CHUNK = 128
PER_W = TOTAL // NW
NCHUNK = PER_W // CHUNK
NBUF = 2

assert PER_W % CHUNK == 0


def _body(ids_hbm, table_hbm, out_hbm, idx_v, rows_v, gsem):
    wid = lax.axis_index("s") * NC + lax.axis_index("c")
    chunk0 = wid * NCHUNK

    pltpu.sync_copy(ids_hbm.at[pl.ds(chunk0, NCHUNK)], idx_v)

    def start_gather(c, b):
        pltpu.async_copy(table_hbm.at[idx_v.at[c]], rows_v.at[b], gsem)

    for b in range(NBUF):
        start_gather(b, b)

    def group(g, with_next):
        for b in range(NBUF):
            c = g * NBUF + b
            pltpu.make_async_copy(table_hbm.at[idx_v.at[c]], rows_v.at[b],
                                  gsem).wait()
            pltpu.sync_copy(rows_v.at[b],
                            out_hbm.at[pl.ds((chunk0 + c) * CHUNK, CHUNK)])
            if with_next:
                start_gather(c + NBUF, b)

    pl.loop(0, NCHUNK // NBUF - 1)(lambda g: group(g, True))
    group(NCHUNK // NBUF - 1, False)


def kernel(ids, table):
    flat_ids = ids.reshape(-1).astype(jnp.int32)
    ids2d = flat_ids.reshape(TOTAL // CHUNK, CHUNK)

    mesh = plsc.VectorSubcoreMesh(core_axis_name="c", subcore_axis_name="s")
    out = pl.kernel(
        _body,
        out_type=jax.ShapeDtypeStruct((TOTAL, EMB_DIM), jnp.float32),
        mesh=mesh,
        scratch_types=[
            pltpu.VMEM((NCHUNK, CHUNK), jnp.int32),
            pltpu.VMEM((NBUF, CHUNK, EMB_DIM), jnp.float32),
            pltpu.SemaphoreType.DMA,
        ],
        compiler_params=pltpu.CompilerParams(use_tc_tiling_on_sc=False),
    )(ids2d, table)
    return out.reshape(BATCH, FIELDS, EMB_DIM)


# speedup vs baseline: 1.5757x; 1.0345x over previous
import functools

import jax
import jax.numpy as jnp
from jax import lax
from jax.experimental import pallas as pl
from jax.experimental.pallas import tpu as pltpu
from jax.experimental.pallas import tpu_sc as plsc

VOCAB = 1000000
EMB_DIM = 32
BATCH = 16384
FIELDS = 26
TOTAL = BATCH * FIELDS

_info = plsc.get_sparse_core_info()
NC, NS = _info.num_cores, _info.num_subcores
NW = NC * NS

CHUNK = 832
PER_W = TOTAL // NW
NCHUNK = PER_W // CHUNK
NBUF = 2

assert PER_W % CHUNK == 0
assert NCHUNK % NBUF == 0
assert CHUNK % 8 == 0


def _body(ids_hbm, table_hbm, out_hbm, idx_v, rows_v, gsem):
    wid = lax.axis_index("s") * NC + lax.axis_index("c")
    chunk0 = wid * NCHUNK

    pltpu.sync_copy(ids_hbm.at[pl.ds(chunk0, NCHUNK)], idx_v)

    def start_gather(c, b):
        pltpu.async_copy(table_hbm.at[idx_v.at[c]], rows_v.at[b], gsem)

    for b in range(NBUF):
        start_gather(b, b)

    def group(g, with_next):
        for b in range(NBUF):
            c = g * NBUF + b
            pltpu.make_async_copy(table_hbm.at[idx_v.at[c]], rows_v.at[b],
                                  gsem).wait()
            pltpu.sync_copy(rows_v.at[b],
                            out_hbm.at[pl.ds((chunk0 + c) * CHUNK, CHUNK)])
            if with_next:
                start_gather(c + NBUF, b)

    pl.loop(0, NCHUNK // NBUF - 1)(lambda g: group(g, True))
    group(NCHUNK // NBUF - 1, False)


def kernel(ids, table):
    flat_ids = ids.reshape(-1).astype(jnp.int32)
    ids2d = flat_ids.reshape(TOTAL // CHUNK, CHUNK)

    mesh = plsc.VectorSubcoreMesh(core_axis_name="c", subcore_axis_name="s")
    out = pl.kernel(
        _body,
        out_type=jax.ShapeDtypeStruct((TOTAL, EMB_DIM), jnp.float32),
        mesh=mesh,
        scratch_types=[
            pltpu.VMEM((NCHUNK, CHUNK), jnp.int32),
            pltpu.VMEM((NBUF, CHUNK, EMB_DIM), jnp.float32),
            pltpu.SemaphoreType.DMA,
        ],
        compiler_params=pltpu.CompilerParams(use_tc_tiling_on_sc=False),
    )(ids2d, table)
    return out.reshape(BATCH, FIELDS, EMB_DIM)
